# baseline (speedup 1.0000x reference)
import dataclasses
import functools

import jax
import jax.numpy as jnp
from jax import lax
from jax.experimental import pallas as pl
from jax.experimental.pallas import tpu as pltpu
from jax.experimental.pallas import tpu_sc as plsc

N = 10000
D = 128
E = 320000
NUM_LAYERS = 3

NC = 2
NS = 16
TILES = NC * NS
C = 128
NCH = 80
BLK = 8
NBLK = NCH // BLK
EPT = NCH * C
E_PAD = TILES * EPT
LANES = 16

STRIPE = 624
LAST_FLUSH = N - (NS - 1) * STRIPE


def _sc_layer(x, rows3d, cols3d, vals3d):
    mesh = plsc.VectorSubcoreMesh(core_axis_name="c", subcore_axis_name="s")

    cp = pltpu.CompilerParams()
    if "needs_layout_passes" in pltpu.CompilerParams.__dataclass_fields__:
        cp = dataclasses.replace(cp, needs_layout_passes=False)

    @functools.partial(
        pl.kernel,
        out_type=jax.ShapeDtypeStruct((NC, N, D), jnp.float32),
        mesh=mesh,
        compiler_params=cp,
        scratch_types=[
            pltpu.VMEM_SHARED((N, D), jnp.float32),
            pltpu.VMEM((BLK, C), jnp.int32),
            pltpu.VMEM((BLK, C), jnp.int32),
            pltpu.VMEM((BLK, C), jnp.int32),
            pltpu.VMEM((BLK, C), jnp.int32),
            pltpu.VMEM((BLK, C), jnp.float32),
            pltpu.VMEM((BLK, C), jnp.float32),
            pltpu.VMEM((C, D), jnp.float32),
            pltpu.VMEM((C, D), jnp.float32),
            pltpu.SemaphoreType.DMA,
            pltpu.SemaphoreType.DMA,
        ],
    )
    def layer(x_hbm, rows_hbm, cols_hbm, vals_hbm, out_hbm,
              acc, rows_b0, rows_b1, cols_b0, cols_b1, vals_b0, vals_b1,
              g0, g1, gs0, gs1):
        cid = lax.axis_index("c")
        sid = lax.axis_index("s")
        tid = cid * NS + sid

        rows_blk = (rows_b0, rows_b1)
        cols_blk = (cols_b0, cols_b1)
        vals_blk = (vals_b0, vals_b1)
        g = (g0, g1)
        gsem = (gs0, gs1)

        zv = jnp.zeros((LANES,), jnp.float32)

        @pl.loop(0, C)
        def _(r):
            for k in range(D // LANES):
                g0[r, pl.ds(k * LANES, LANES)] = zv

        @pl.loop(0, 5)
        def _(b):
            pltpu.sync_copy(g0, acc.at[pl.ds(sid * STRIPE + b * C, C)])

        plsc.subcore_barrier()

        def stage(blk_idx, pb):
            off = pl.multiple_of(blk_idx * BLK, BLK)
            pltpu.sync_copy(rows_hbm.at[tid, pl.ds(off, BLK)], rows_blk[pb])
            pltpu.sync_copy(cols_hbm.at[tid, pl.ds(off, BLK)], cols_blk[pb])
            pltpu.sync_copy(vals_hbm.at[tid, pl.ds(off, BLK)], vals_blk[pb])

        def start_gather(pb, c, b):
            pltpu.async_copy(x_hbm.at[cols_blk[pb].at[c]], g[b], gsem[b])

        def wait_gather(pb, c, b):
            pltpu.make_async_copy(
                x_hbm.at[cols_blk[pb].at[c]], g[b], gsem[b]).wait()

        def scale(pb, c, b):
            @pl.loop(0, C, step=8)
            def _(i0):
                gb = g[b]
                ws = [plsc.load_gather(
                    vals_blk[pb],
                    [jnp.full((LANES,), c, jnp.int32),
                     jnp.full((LANES,), i0 + u, jnp.int32)])
                    for u in range(8)]
                for u in range(8):
                    for k in range(D // LANES):
                        sl = pl.ds(k * LANES, LANES)
                        gb[i0 + u, sl] = gb[i0 + u, sl] * ws[u]

        stage(0, 0)
        start_gather(0, 0, 0)

        @pl.loop(0, NBLK, step=2)
        def _(blk):
            for pb in range(2):
                bi = blk + pb
                for c in range(BLK):
                    b = c % 2
                    wait_gather(pb, c, b)
                    if c < BLK - 1:
                        start_gather(pb, c + 1, 1 - b)
                    else:
                        npb = (pb + 1) % 2

                        @pl.when(bi < NBLK - 1)
                        def _():
                            stage(bi + 1, npb)
                            start_gather(npb, 0, 1 - b)

                    scale(pb, c, b)
                    pltpu.sync_copy(g[b], acc.at[rows_blk[pb].at[c]],
                                    add=True)

        plsc.subcore_barrier()

        @pl.when(sid < NS - 1)
        def _():
            pltpu.sync_copy(acc.at[pl.ds(sid * STRIPE, STRIPE)],
                            out_hbm.at[cid, pl.ds(sid * STRIPE, STRIPE)])

        @pl.when(sid == NS - 1)
        def _():
            pltpu.sync_copy(acc.at[pl.ds((NS - 1) * STRIPE, LAST_FLUSH)],
                            out_hbm.at[cid, pl.ds((NS - 1) * STRIPE, LAST_FLUSH)])

    return layer(x, rows3d, cols3d, vals3d)


def _tc_add(parts):
    def body(p_ref, o_ref):
        o_ref[...] = p_ref[0] + p_ref[1]

    return pl.pallas_call(
        body,
        out_shape=jax.ShapeDtypeStruct((N, D), jnp.float32),
        grid=(10,),
        in_specs=[pl.BlockSpec((2, N // 10, D), lambda i: (0, i, 0))],
        out_specs=pl.BlockSpec((N // 10, D), lambda i: (i, 0)),
    )(parts)


@jax.jit
def kernel(x0, adj_rows, adj_cols, adj_vals):
    pad = E_PAD - E
    spread = (jnp.arange(pad, dtype=jnp.int32) * 8) % N
    rows3d = jnp.concatenate(
        [adj_rows.astype(jnp.int32), spread]).reshape(TILES, NCH, C)
    cols3d = jnp.concatenate(
        [adj_cols.astype(jnp.int32), spread]).reshape(TILES, NCH, C)
    vals3d = jnp.pad(adj_vals.astype(jnp.float32), (0, pad)).reshape(
        TILES, NCH, C)

    xs = [x0]
    x = x0
    for _ in range(NUM_LAYERS):
        parts = _sc_layer(x, rows3d, cols3d, vals3d)
        x = _tc_add(parts)
        xs.append(x)
    return tuple(xs)


# speedup vs baseline: 1.5369x; 1.5369x over previous
import dataclasses
import functools

import jax
import jax.numpy as jnp
from jax import lax
from jax.experimental import pallas as pl
from jax.experimental.pallas import tpu as pltpu
from jax.experimental.pallas import tpu_sc as plsc

N = 10000
D = 128
E = 320000
NUM_LAYERS = 3

NC = 2
NS = 16
TILES = NC * NS
C = 80
NCH = 128
BLK = 8
NBLK = NCH // BLK
EPT = NCH * C
E_PAD = TILES * EPT
LANES = 16

STRIPE = 624
LAST_FLUSH = N - (NS - 1) * STRIPE


def _sc_layer(x, rows3d, cols3d, vals3d):
    mesh = plsc.VectorSubcoreMesh(core_axis_name="c", subcore_axis_name="s")

    cp = pltpu.CompilerParams()
    if "needs_layout_passes" in pltpu.CompilerParams.__dataclass_fields__:
        cp = dataclasses.replace(cp, needs_layout_passes=False)

    @functools.partial(
        pl.kernel,
        out_type=jax.ShapeDtypeStruct((NC, N, D), jnp.float32),
        mesh=mesh,
        compiler_params=cp,
        scratch_types=[
            pltpu.VMEM_SHARED((N, D), jnp.float32),
            pltpu.VMEM((BLK, C), jnp.int32),
            pltpu.VMEM((BLK, C), jnp.int32),
            pltpu.VMEM((BLK, C), jnp.int32),
            pltpu.VMEM((BLK, C), jnp.int32),
            pltpu.VMEM((BLK, C), jnp.float32),
            pltpu.VMEM((BLK, C), jnp.float32),
            pltpu.VMEM((C, D), jnp.float32),
            pltpu.VMEM((C, D), jnp.float32),
            pltpu.VMEM((C, D), jnp.float32),
            pltpu.VMEM((C, D), jnp.float32),
            pltpu.SemaphoreType.DMA,
            pltpu.SemaphoreType.DMA,
            pltpu.SemaphoreType.DMA,
            pltpu.SemaphoreType.DMA,
            pltpu.SemaphoreType.DMA,
            pltpu.SemaphoreType.DMA,
            pltpu.SemaphoreType.DMA,
            pltpu.SemaphoreType.DMA,
        ],
    )
    def layer(x_hbm, rows_hbm, cols_hbm, vals_hbm, out_hbm,
              acc, rows_b0, rows_b1, cols_b0, cols_b1, vals_b0, vals_b1,
              g0, g1, g2, g3, gs0, gs1, gs2, gs3, ss0, ss1, ss2, ss3):
        cid = lax.axis_index("c")
        sid = lax.axis_index("s")
        tid = cid * NS + sid

        rows_blk = (rows_b0, rows_b1)
        cols_blk = (cols_b0, cols_b1)
        vals_blk = (vals_b0, vals_b1)
        g = (g0, g1, g2, g3)
        gsem = (gs0, gs1, gs2, gs3)
        ssem = (ss0, ss1, ss2, ss3)

        zv = jnp.zeros((LANES,), jnp.float32)

        @pl.loop(0, C)
        def _(r):
            for k in range(D // LANES):
                g0[r, pl.ds(k * LANES, LANES)] = zv

        @pl.loop(0, 8)
        def _(b):
            pltpu.sync_copy(g0, acc.at[pl.ds(sid * STRIPE + b * C, C)])

        plsc.subcore_barrier()

        def stage(blk_idx, pb):
            off = pl.multiple_of(blk_idx * BLK, BLK)
            pltpu.sync_copy(rows_hbm.at[tid, pl.ds(off, BLK)], rows_blk[pb])
            pltpu.sync_copy(cols_hbm.at[tid, pl.ds(off, BLK)], cols_blk[pb])
            pltpu.sync_copy(vals_hbm.at[tid, pl.ds(off, BLK)], vals_blk[pb])

        def start_gather(pb, c, b):
            pltpu.async_copy(x_hbm.at[cols_blk[pb].at[c]], g[b], gsem[b])

        def wait_gather(pb, c, b):
            pltpu.make_async_copy(
                x_hbm.at[cols_blk[pb].at[c]], g[b], gsem[b]).wait()

        def start_scatter(pb, c, b):
            pltpu.async_copy(g[b], acc.at[rows_blk[pb].at[c]], ssem[b],
                             add=True)

        def wait_scatter(pb, c, b):
            pltpu.make_async_copy(
                g[b], acc.at[rows_blk[pb].at[c]], ssem[b]).wait()

        def scale(pb, c, b):
            @pl.loop(0, C, step=4)
            def _(i0):
                gb = g[b]
                ws = [plsc.load_gather(
                    vals_blk[pb],
                    [jnp.full((LANES,), c, jnp.int32),
                     jnp.full((LANES,), i0 + u, jnp.int32)])
                    for u in range(4)]
                for u in range(4):
                    for k in range(D // LANES):
                        sl = pl.ds(k * LANES, LANES)
                        gb[i0 + u, sl] = gb[i0 + u, sl] * ws[u]

        stage(0, 0)
        start_gather(0, 0, 0)
        start_gather(0, 1, 1)

        @pl.loop(0, NBLK, step=2)
        def _(blk):
            for pb in range(2):
                bi = blk + pb
                npb = (pb + 1) % 2
                for c in range(BLK):
                    b = c % 4
                    nb = (c + 2) % 4
                    wait_gather(pb, c, b)
                    scale(pb, c, b)
                    start_scatter(pb, c, b)
                    if c < 2:
                        @pl.when(bi > 0)
                        def _():
                            wait_scatter(npb, BLK - 2 + c, nb)

                        start_gather(pb, c + 2, nb)
                    elif c < BLK - 2:
                        wait_scatter(pb, c - 2, nb)
                        start_gather(pb, c + 2, nb)
                    else:
                        @pl.when(bi < NBLK - 1)
                        def _():
                            wait_scatter(pb, c - 2, nb)
                            start_gather(npb, c - (BLK - 2), nb)

                    if c == 5:
                        @pl.when(bi < NBLK - 1)
                        def _():
                            stage(bi + 1, npb)

        wait_scatter((NBLK - 1) % 2, BLK - 2, 2)
        wait_scatter((NBLK - 1) % 2, BLK - 1, 3)

        plsc.subcore_barrier()

        @pl.when(sid < NS - 1)
        def _():
            pltpu.sync_copy(acc.at[pl.ds(sid * STRIPE, STRIPE)],
                            out_hbm.at[cid, pl.ds(sid * STRIPE, STRIPE)])

        @pl.when(sid == NS - 1)
        def _():
            pltpu.sync_copy(acc.at[pl.ds((NS - 1) * STRIPE, LAST_FLUSH)],
                            out_hbm.at[cid, pl.ds((NS - 1) * STRIPE, LAST_FLUSH)])

    return layer(x, rows3d, cols3d, vals3d)


def _tc_add(parts):
    def body(p_ref, o_ref):
        o_ref[...] = p_ref[0] + p_ref[1]

    return pl.pallas_call(
        body,
        out_shape=jax.ShapeDtypeStruct((N, D), jnp.float32),
        grid=(10,),
        in_specs=[pl.BlockSpec((2, N // 10, D), lambda i: (0, i, 0))],
        out_specs=pl.BlockSpec((N // 10, D), lambda i: (i, 0)),
    )(parts)


@jax.jit
def kernel(x0, adj_rows, adj_cols, adj_vals):
    pad = E_PAD - E
    spread = (jnp.arange(pad, dtype=jnp.int32) * 8) % N
    rows3d = jnp.concatenate(
        [adj_rows.astype(jnp.int32), spread]).reshape(TILES, NCH, C)
    cols3d = jnp.concatenate(
        [adj_cols.astype(jnp.int32), spread]).reshape(TILES, NCH, C)
    vals3d = jnp.pad(adj_vals.astype(jnp.float32), (0, pad)).reshape(
        TILES, NCH, C)

    xs = [x0]
    x = x0
    for _ in range(NUM_LAYERS):
        parts = _sc_layer(x, rows3d, cols3d, vals3d)
        x = _tc_add(parts)
        xs.append(x)
    return tuple(xs)


# speedup vs baseline: 1.5620x; 1.0163x over previous
import dataclasses
import functools

import jax
import jax.numpy as jnp
from jax import lax
from jax.experimental import pallas as pl
from jax.experimental.pallas import tpu as pltpu
from jax.experimental.pallas import tpu_sc as plsc

N = 10000
D = 128
E = 320000
NUM_LAYERS = 3

NC = 2
NS = 16
TILES = NC * NS
C = 80
NCH = 128
BLK = 8
NBLK = NCH // BLK
EPT = NCH * C
E_PAD = TILES * EPT
LANES = 16

STRIPE = 624
LAST_FLUSH = N - (NS - 1) * STRIPE


def _sc_layer(x, rows3d, cols3d, vals3d):
    mesh = plsc.VectorSubcoreMesh(core_axis_name="c", subcore_axis_name="s")

    cp = pltpu.CompilerParams()
    if "needs_layout_passes" in pltpu.CompilerParams.__dataclass_fields__:
        cp = dataclasses.replace(cp, needs_layout_passes=False)

    @functools.partial(
        pl.kernel,
        out_type=jax.ShapeDtypeStruct((NC, N, D), jnp.float32),
        mesh=mesh,
        compiler_params=cp,
        scratch_types=[
            pltpu.VMEM_SHARED((N, D), jnp.float32),
            pltpu.VMEM((BLK, C), jnp.int32),
            pltpu.VMEM((BLK, C), jnp.int32),
            pltpu.VMEM((BLK, C), jnp.int32),
            pltpu.VMEM((BLK, C), jnp.int32),
            pltpu.VMEM((BLK, C), jnp.float32),
            pltpu.VMEM((BLK, C), jnp.float32),
            pltpu.VMEM((C, D), jnp.float32),
            pltpu.VMEM((C, D), jnp.float32),
            pltpu.VMEM((C, D), jnp.float32),
            pltpu.VMEM((C, D), jnp.float32),
            pltpu.SemaphoreType.DMA,
            pltpu.SemaphoreType.DMA,
            pltpu.SemaphoreType.DMA,
            pltpu.SemaphoreType.DMA,
            pltpu.SemaphoreType.DMA,
            pltpu.SemaphoreType.DMA,
            pltpu.SemaphoreType.DMA,
            pltpu.SemaphoreType.DMA,
        ],
    )
    def layer(x_hbm, rows_hbm, cols_hbm, vals_hbm, out_hbm,
              acc, rows_b0, rows_b1, cols_b0, cols_b1, vals_b0, vals_b1,
              g0, g1, g2, g3, gs0, gs1, gs2, gs3, ss0, ss1, ss2, ss3):
        cid = lax.axis_index("c")
        sid = lax.axis_index("s")
        tid = cid * NS + sid

        rows_blk = (rows_b0, rows_b1)
        cols_blk = (cols_b0, cols_b1)
        vals_blk = (vals_b0, vals_b1)
        g = (g0, g1, g2, g3)
        gsem = (gs0, gs1, gs2, gs3)
        ssem = (ss0, ss1, ss2, ss3)

        zv = jnp.zeros((LANES,), jnp.float32)

        @pl.loop(0, C)
        def _(r):
            for k in range(D // LANES):
                g0[r, pl.ds(k * LANES, LANES)] = zv

        @pl.loop(0, 8)
        def _(b):
            pltpu.sync_copy(g0, acc.at[pl.ds(sid * STRIPE + b * C, C)])

        plsc.subcore_barrier()

        def stage(blk_idx, pb):
            off = pl.multiple_of(blk_idx * BLK, BLK)
            pltpu.sync_copy(rows_hbm.at[tid, pl.ds(off, BLK)], rows_blk[pb])
            pltpu.sync_copy(cols_hbm.at[tid, pl.ds(off, BLK)], cols_blk[pb])
            pltpu.sync_copy(vals_hbm.at[tid, pl.ds(off, BLK)], vals_blk[pb])

        def start_gather(pb, c, b):
            pltpu.async_copy(x_hbm.at[cols_blk[pb].at[c]], g[b], gsem[b])

        def wait_gather(pb, c, b):
            pltpu.make_async_copy(
                x_hbm.at[cols_blk[pb].at[c]], g[b], gsem[b]).wait()

        def start_scatter(pb, c, b):
            pltpu.async_copy(g[b], acc.at[rows_blk[pb].at[c]], ssem[b],
                             add=True)

        def wait_scatter(pb, c, b):
            pltpu.make_async_copy(
                g[b], acc.at[rows_blk[pb].at[c]], ssem[b]).wait()

        def scale(pb, c, b):
            @plsc.parallel_loop(0, C, step=4)
            def _(i0):
                gb = g[b]
                ws = [plsc.load_gather(
                    vals_blk[pb],
                    [jnp.full((LANES,), c, jnp.int32),
                     jnp.full((LANES,), i0 + u, jnp.int32)])
                    for u in range(4)]
                for u in range(4):
                    for k in range(D // LANES):
                        sl = pl.ds(k * LANES, LANES)
                        gb[i0 + u, sl] = gb[i0 + u, sl] * ws[u]

        stage(0, 0)
        start_gather(0, 0, 0)
        start_gather(0, 1, 1)

        @pl.loop(0, NBLK, step=2)
        def _(blk):
            for pb in range(2):
                bi = blk + pb
                npb = (pb + 1) % 2
                for c in range(BLK):
                    b = c % 4
                    nb = (c + 2) % 4
                    wait_gather(pb, c, b)
                    scale(pb, c, b)
                    start_scatter(pb, c, b)
                    if c < 2:
                        @pl.when(bi > 0)
                        def _():
                            wait_scatter(npb, BLK - 2 + c, nb)

                        start_gather(pb, c + 2, nb)
                    elif c < BLK - 2:
                        wait_scatter(pb, c - 2, nb)
                        start_gather(pb, c + 2, nb)
                    else:
                        @pl.when(bi < NBLK - 1)
                        def _():
                            wait_scatter(pb, c - 2, nb)
                            start_gather(npb, c - (BLK - 2), nb)

                    if c == 5:
                        @pl.when(bi < NBLK - 1)
                        def _():
                            stage(bi + 1, npb)

        wait_scatter((NBLK - 1) % 2, BLK - 2, 2)
        wait_scatter((NBLK - 1) % 2, BLK - 1, 3)

        plsc.subcore_barrier()

        @pl.when(sid < NS - 1)
        def _():
            pltpu.sync_copy(acc.at[pl.ds(sid * STRIPE, STRIPE)],
                            out_hbm.at[cid, pl.ds(sid * STRIPE, STRIPE)])

        @pl.when(sid == NS - 1)
        def _():
            pltpu.sync_copy(acc.at[pl.ds((NS - 1) * STRIPE, LAST_FLUSH)],
                            out_hbm.at[cid, pl.ds((NS - 1) * STRIPE, LAST_FLUSH)])

    return layer(x, rows3d, cols3d, vals3d)


def _tc_add(parts):
    def body(p_ref, o_ref):
        o_ref[...] = p_ref[0] + p_ref[1]

    return pl.pallas_call(
        body,
        out_shape=jax.ShapeDtypeStruct((N, D), jnp.float32),
        grid=(10,),
        in_specs=[pl.BlockSpec((2, N // 10, D), lambda i: (0, i, 0))],
        out_specs=pl.BlockSpec((N // 10, D), lambda i: (i, 0)),
    )(parts)


@jax.jit
def kernel(x0, adj_rows, adj_cols, adj_vals):
    pad = E_PAD - E
    spread = (jnp.arange(pad, dtype=jnp.int32) * 8) % N
    rows3d = jnp.concatenate(
        [adj_rows.astype(jnp.int32), spread]).reshape(TILES, NCH, C)
    cols3d = jnp.concatenate(
        [adj_cols.astype(jnp.int32), spread]).reshape(TILES, NCH, C)
    vals3d = jnp.pad(adj_vals.astype(jnp.float32), (0, pad)).reshape(
        TILES, NCH, C)

    xs = [x0]
    x = x0
    for _ in range(NUM_LAYERS):
        parts = _sc_layer(x, rows3d, cols3d, vals3d)
        x = _tc_add(parts)
        xs.append(x)
    return tuple(xs)
